# baseline (speedup 1.0000x reference)
import functools

import jax
import jax.numpy as jnp
from jax import lax
from jax.experimental import pallas as pl
from jax.experimental.pallas import tpu as pltpu
from jax.experimental.pallas import tpu_sc as plsc

H = 128
EPS = 1e-05
BB = 256
NBUF = 3


def _sc_gather(table, idx_pad, n_pad):
    info = plsc.get_sparse_core_info()
    nw = info.num_cores * info.num_subcores
    per = n_pad // nw

    @functools.partial(
        pl.kernel,
        mesh=plsc.VectorSubcoreMesh(core_axis_name="c", subcore_axis_name="s"),
        out_type=jax.ShapeDtypeStruct((n_pad, H), jnp.float32),
        scratch_types=[
            pltpu.VMEM((per,), jnp.int32),
            pltpu.VMEM((per, H), jnp.float32),
            pltpu.SemaphoreType.DMA,
        ],
    )
    def gk(table_hbm, idx_hbm, out_hbm, idx_v, rows_v, sem):
        wid = lax.axis_index("s") * info.num_cores + lax.axis_index("c")
        base = wid * per
        pltpu.sync_copy(idx_hbm.at[pl.ds(base, per)], idx_v)
        pltpu.async_copy(table_hbm.at[idx_v], rows_v, sem).wait()
        pltpu.sync_copy(rows_v, out_hbm.at[pl.ds(base, per)])

    return gk(table, idx_pad)


def _main_body(l, emb_ref, mask_ref, lnw_ref, lnb_ref, bias_ref, w_ref,
               x_ref, out_ref, *rest):
    bufs, sems = rest[:NBUF], rest[NBUF:]
    cl = mask_ref.shape[0]

    e = emb_ref[0:cl, :]
    mu = jnp.mean(e, axis=1, keepdims=True)
    d = e - mu
    var = jnp.mean(d * d, axis=1, keepdims=True)
    ln = d * lax.rsqrt(var + EPS) * lnw_ref[:] + lnb_ref[:]
    m = mask_ref[:]
    s = jnp.sum((ln * m).reshape(cl // l, l, H), axis=1)
    cnt = jnp.sum(m.reshape(cl // l, l, 1), axis=1)
    col = s / cnt

    dn = (((1,), (1,)), ((), ()))
    p = lax.dot_general(col, w_ref[:], dn,
                        precision=lax.Precision.HIGHEST,
                        preferred_element_type=jnp.float32)
    q = lax.dot_general(bias_ref[:], w_ref[:], dn,
                        precision=lax.Precision.HIGHEST,
                        preferred_element_type=jnp.float32)

    cb = bufs[0].shape[0]
    nch = x_ref.shape[0] // cb
    for j in range(nch):
        r = j % NBUF
        if j >= NBUF:
            pltpu.make_async_copy(
                bufs[r], out_ref.at[pl.ds((j - NBUF) * cb, cb)],
                sems[r]).wait()
        x = x_ref[pl.ds(j * cb, cb), :]
        bufs[r][:] = x[:, :, None] * p + q
        pltpu.make_async_copy(
            bufs[r], out_ref.at[pl.ds(j * cb, cb)], sems[r]).start()
    for j in range(max(nch - NBUF, 0), nch):
        r = j % NBUF
        pltpu.make_async_copy(
            bufs[r], out_ref.at[pl.ds(j * cb, cb)], sems[r]).wait()


def kernel(x_num, num_col_input_ids, num_att_mask, word_emb, ln_w, ln_b, num_bias, align_W):
    b, c = x_num.shape
    l = num_col_input_ids.shape[1]
    cl = c * l
    n_pad = ((cl + 255) // 256) * 256

    ids = num_col_input_ids.reshape(cl).astype(jnp.int32)
    ids = jnp.pad(ids, (0, n_pad - cl))
    emb = _sc_gather(word_emb, ids, n_pad)

    mask = num_att_mask.reshape(cl, 1).astype(jnp.float32)
    out = pl.pallas_call(
        functools.partial(_main_body, l),
        out_specs=pl.BlockSpec(memory_space=pl.ANY),
        out_shape=jax.ShapeDtypeStruct((b, c, H), jnp.float32),
        scratch_shapes=(
            [pltpu.VMEM((BB, c, H), jnp.float32) for _ in range(NBUF)]
            + [pltpu.SemaphoreType.DMA for _ in range(NBUF)]
        ),
    )(emb, mask, ln_w.reshape(1, H), ln_b.reshape(1, H),
      num_bias.reshape(1, H), align_W, x_num)

    attention_mask = jnp.ones((b, c), dtype=jnp.float32)
    return (out, attention_mask)


# speedup vs baseline: 1.0201x; 1.0201x over previous
import functools

import jax
import jax.numpy as jnp
from jax import lax
from jax.experimental import pallas as pl
from jax.experimental.pallas import tpu as pltpu
from jax.experimental.pallas import tpu_sc as plsc

H = 128
EPS = 1e-05
BB = 128
NBUF = 4


def _sc_gather(table, idx_pad, n_pad):
    info = plsc.get_sparse_core_info()
    nw = info.num_cores * info.num_subcores
    per = n_pad // nw

    @functools.partial(
        pl.kernel,
        mesh=plsc.VectorSubcoreMesh(core_axis_name="c", subcore_axis_name="s"),
        out_type=jax.ShapeDtypeStruct((n_pad, H), jnp.float32),
        scratch_types=[
            pltpu.VMEM((per,), jnp.int32),
            pltpu.VMEM((per, H), jnp.float32),
            pltpu.SemaphoreType.DMA,
        ],
    )
    def gk(table_hbm, idx_hbm, out_hbm, idx_v, rows_v, sem):
        wid = lax.axis_index("s") * info.num_cores + lax.axis_index("c")
        base = wid * per
        pltpu.sync_copy(idx_hbm.at[pl.ds(base, per)], idx_v)
        pltpu.async_copy(table_hbm.at[idx_v], rows_v, sem).wait()
        pltpu.sync_copy(rows_v, out_hbm.at[pl.ds(base, per)])

    return gk(table, idx_pad)


def _main_body(l, emb_ref, mask_ref, lnw_ref, lnb_ref, bias_ref, w_ref,
               x_ref, out_ref, *rest):
    bufs, sems = rest[:NBUF], rest[NBUF:]
    cl = mask_ref.shape[0]

    e = emb_ref[0:cl, :]
    mu = jnp.mean(e, axis=1, keepdims=True)
    d = e - mu
    var = jnp.mean(d * d, axis=1, keepdims=True)
    ln = d * lax.rsqrt(var + EPS) * lnw_ref[:] + lnb_ref[:]
    m = mask_ref[:]
    s = jnp.sum((ln * m).reshape(cl // l, l, H), axis=1)
    cnt = jnp.sum(m.reshape(cl // l, l, 1), axis=1)
    col = s / cnt

    dn = (((1,), (1,)), ((), ()))
    p = lax.dot_general(col, w_ref[:], dn,
                        precision=lax.Precision.HIGHEST,
                        preferred_element_type=jnp.float32)
    q = lax.dot_general(bias_ref[:], w_ref[:], dn,
                        precision=lax.Precision.HIGHEST,
                        preferred_element_type=jnp.float32)

    cb = bufs[0].shape[0]
    nch = x_ref.shape[0] // cb
    for j in range(nch):
        r = j % NBUF
        if j >= NBUF:
            pltpu.make_async_copy(
                bufs[r], out_ref.at[pl.ds((j - NBUF) * cb, cb)],
                sems[r]).wait()
        x = x_ref[pl.ds(j * cb, cb), :]
        bufs[r][:] = x[:, :, None] * p + q
        pltpu.make_async_copy(
            bufs[r], out_ref.at[pl.ds(j * cb, cb)], sems[r]).start()
    for j in range(max(nch - NBUF, 0), nch):
        r = j % NBUF
        pltpu.make_async_copy(
            bufs[r], out_ref.at[pl.ds(j * cb, cb)], sems[r]).wait()


def kernel(x_num, num_col_input_ids, num_att_mask, word_emb, ln_w, ln_b, num_bias, align_W):
    b, c = x_num.shape
    l = num_col_input_ids.shape[1]
    cl = c * l
    n_pad = ((cl + 255) // 256) * 256

    ids = num_col_input_ids.reshape(cl).astype(jnp.int32)
    ids = jnp.pad(ids, (0, n_pad - cl))
    emb = _sc_gather(word_emb, ids, n_pad)

    mask = num_att_mask.reshape(cl, 1).astype(jnp.float32)
    out = pl.pallas_call(
        functools.partial(_main_body, l),
        out_specs=pl.BlockSpec(memory_space=pl.ANY),
        out_shape=jax.ShapeDtypeStruct((b, c, H), jnp.float32),
        scratch_shapes=(
            [pltpu.VMEM((BB, c, H), jnp.float32) for _ in range(NBUF)]
            + [pltpu.SemaphoreType.DMA for _ in range(NBUF)]
        ),
    )(emb, mask, ln_w.reshape(1, H), ln_b.reshape(1, H),
      num_bias.reshape(1, H), align_W, x_num)

    attention_mask = jnp.ones((b, c), dtype=jnp.float32)
    return (out, attention_mask)
